# baseline (speedup 1.0000x reference)
import functools

import jax
import jax.numpy as jnp
from jax.experimental import pallas as pl

_EPS = 1e-08
_N_ROWS = 16384
_N_FEAT = 100
_N_CAT = 6
_BT = 2048


def _body(x_ref, wt_ref, b_ref, g_ref, elig_ref, a_ref, ent_ref):
    x1 = jnp.dot(x_ref[...], wt_ref[...]) + b_ref[...]
    m = jnp.max(x1, axis=-1, keepdims=True)
    e = jnp.exp(x1 - m)
    s = jnp.sum(e, axis=-1, keepdims=True)
    p = e / s
    logp = (x1 - m) - jnp.log(s)

    z = x1 + g_ref[...]
    zm = jnp.max(z, axis=-1, keepdims=True)
    iota = jax.lax.broadcasted_iota(jnp.int32, z.shape, 1)
    cand = jnp.where(z == zm, iota, _N_CAT)
    idx = jnp.min(cand, axis=-1, keepdims=True)
    a = (iota == idx).astype(jnp.float32)

    elig_ref[...] = jnp.sum(a * logp, axis=-1, keepdims=True)
    a_ref[...] = a
    pe = p + _EPS
    part = jnp.sum(-pe * jnp.log(pe)).reshape(1, 1)
    @pl.when(pl.program_id(0) == 0)
    def _init():
        ent_ref[...] = part
    @pl.when(pl.program_id(0) != 0)
    def _acc():
        ent_ref[...] += part


@functools.partial(jax.jit, static_argnums=())
def kernel(x, W, b):
    g = jax.random.gumbel(jax.random.key(42), (_N_ROWS, _N_CAT), jnp.float32)
    elig, a, ent = pl.pallas_call(
        _body,
        grid=(_N_ROWS // _BT,),
        in_specs=[
            pl.BlockSpec((_BT, _N_FEAT), lambda i: (i, 0)),
            pl.BlockSpec((_N_FEAT, _N_CAT), lambda i: (0, 0)),
            pl.BlockSpec((1, _N_CAT), lambda i: (0, 0)),
            pl.BlockSpec((_BT, _N_CAT), lambda i: (i, 0)),
        ],
        out_specs=[
            pl.BlockSpec((_BT, 1), lambda i: (i, 0)),
            pl.BlockSpec((_BT, _N_CAT), lambda i: (i, 0)),
            pl.BlockSpec((1, 1), lambda i: (0, 0)),
        ],
        out_shape=[
            jax.ShapeDtypeStruct((_N_ROWS, 1), jnp.float32),
            jax.ShapeDtypeStruct((_N_ROWS, _N_CAT), jnp.float32),
            jax.ShapeDtypeStruct((1, 1), jnp.float32),
        ],
    )(x, W.T, b[None, :], g)
    return elig[:, 0], a, ent[0, 0]


# speedup vs baseline: 1.3569x; 1.3569x over previous
import functools

import jax
import jax.numpy as jnp
from jax import lax
from jax.experimental import pallas as pl
from jax.experimental.pallas import tpu as pltpu
from jax.experimental.pallas import tpu_sc as plsc

_EPS = 1e-08
_N_ROWS = 16384
_N_FEAT = 100
_N_CAT = 6
_BT = 2048
_NW = 32
_RPW = _N_ROWS // _NW
_NG = _RPW // 16

_LN2 = 0.6931471805599453


def _mm_body(w_ref, x_ref, b_ref, g_ref, x1t_ref, z_ref):
    x1t = lax.dot_general(w_ref[...], x_ref[...],
                          (((1,), (1,)), ((), ()))) + b_ref[...]
    x1t_ref[...] = x1t
    z_ref[...] = x1t + g_ref[...]


def _log_poly(s):
    bits = lax.bitcast_convert_type(s, jnp.int32)
    e = ((bits >> 23) - 127).astype(jnp.float32)
    m = lax.bitcast_convert_type((bits & 0x7FFFFF) | 0x3F800000, jnp.float32)
    u = (m - 1.0) / (m + 1.0)
    u2 = u * u
    poly = 2.0 * u * (1.0 + u2 * (1.0 / 3.0 + u2 * (1.0 / 5.0
                      + u2 * (1.0 / 7.0 + u2 * (1.0 / 9.0)))))
    return poly + e * _LN2


def _sc_body(x1t_hbm, z_hbm, elig_hbm, a_hbm, ent_hbm,
             x1_v, z_v, elig_v, a_v, ent_v):
    wid = lax.axis_index("s") * 2 + lax.axis_index("c")
    base = wid * _RPW
    pltpu.sync_copy(x1t_hbm.at[:, pl.ds(base, _RPW)], x1_v)
    pltpu.sync_copy(z_hbm.at[:, pl.ds(base, _RPW)], z_v)

    lane = lax.iota(jnp.int32, 16)

    def group(i, ent_acc):
        off = i * 16
        l = [x1_v[k, pl.ds(off, 16)] for k in range(_N_CAT)]
        z = [z_v[k, pl.ds(off, 16)] for k in range(_N_CAT)]

        m = l[0]
        for k in range(1, _N_CAT):
            m = jnp.maximum(m, l[k])
        e = [jnp.exp(l[k] - m) for k in range(_N_CAT)]
        s = e[0]
        for k in range(1, _N_CAT):
            s = s + e[k]
        r = m + _log_poly(s)
        logp = [l[k] - r for k in range(_N_CAT)]

        best = z[0]
        idx = jnp.zeros((16,), jnp.int32)
        for k in range(1, _N_CAT):
            take = z[k] > best
            best = jnp.maximum(best, z[k])
            idx = jnp.where(take, jnp.full((16,), k, jnp.int32), idx)

        rows6 = (off + lane) * _N_CAT
        elig = jnp.zeros((16,), jnp.float32)
        ent = ent_acc
        inv_s = 1.0 / s
        for k in range(_N_CAT):
            a_k = jnp.where(idx == k, 1.0, 0.0)
            plsc.store_scatter(a_v, [rows6 + k], a_k)
            elig = elig + a_k * logp[k]
            p_k = e[k] * inv_s
            ent = ent + (p_k + _EPS) * logp[k]
        elig_v[pl.ds(off, 16)] = elig
        return ent

    ent_acc = lax.fori_loop(0, _NG, group, jnp.zeros((16,), jnp.float32))
    ent_v[...] = -ent_acc

    pltpu.sync_copy(elig_v, elig_hbm.at[pl.ds(base, _RPW)])
    pltpu.sync_copy(a_v, a_hbm.at[pl.ds(base * _N_CAT, _RPW * _N_CAT)])
    pltpu.sync_copy(ent_v, ent_hbm.at[wid])


def _ent_body(p_ref, o_ref):
    o_ref[...] = jnp.sum(p_ref[...]).reshape(1, 1)


def kernel(x, W, b):
    with jax.ensure_compile_time_eval():
        g = jax.random.gumbel(jax.random.key(42), (_N_ROWS, _N_CAT),
                              jnp.float32)
        gt = g.T.copy()

    x1t, z = pl.pallas_call(
        _mm_body,
        grid=(_N_ROWS // _BT,),
        in_specs=[
            pl.BlockSpec((_N_CAT, _N_FEAT), lambda i: (0, 0)),
            pl.BlockSpec((_BT, _N_FEAT), lambda i: (i, 0)),
            pl.BlockSpec((_N_CAT, 1), lambda i: (0, 0)),
            pl.BlockSpec((_N_CAT, _BT), lambda i: (0, i)),
        ],
        out_specs=[
            pl.BlockSpec((_N_CAT, _BT), lambda i: (0, i)),
            pl.BlockSpec((_N_CAT, _BT), lambda i: (0, i)),
        ],
        out_shape=[
            jax.ShapeDtypeStruct((_N_CAT, _N_ROWS), jnp.float32),
            jax.ShapeDtypeStruct((_N_CAT, _N_ROWS), jnp.float32),
        ],
    )(W, x, b[:, None], gt)

    mesh = plsc.VectorSubcoreMesh(core_axis_name="c", subcore_axis_name="s")
    elig, a, ent_parts = pl.kernel(
        _sc_body,
        out_type=[
            jax.ShapeDtypeStruct((_N_ROWS,), jnp.float32),
            jax.ShapeDtypeStruct((_N_ROWS * _N_CAT,), jnp.float32),
            jax.ShapeDtypeStruct((_NW, 16), jnp.float32),
        ],
        mesh=mesh,
        compiler_params=pltpu.CompilerParams(needs_layout_passes=False),
        scratch_types=[
            pltpu.VMEM((_N_CAT, _RPW), jnp.float32),
            pltpu.VMEM((_N_CAT, _RPW), jnp.float32),
            pltpu.VMEM((_RPW,), jnp.float32),
            pltpu.VMEM((_RPW * _N_CAT,), jnp.float32),
            pltpu.VMEM((16,), jnp.float32),
        ],
    )(x1t, z)

    ent = pl.pallas_call(
        _ent_body,
        in_specs=[pl.BlockSpec((_NW, 16), lambda: (0, 0))],
        out_specs=pl.BlockSpec((1, 1), lambda: (0, 0)),
        out_shape=jax.ShapeDtypeStruct((1, 1), jnp.float32),
    )(ent_parts)

    return elig, a.reshape(_N_ROWS, _N_CAT), ent[0, 0]


# speedup vs baseline: 1.5504x; 1.1426x over previous
import functools

import jax
import jax.numpy as jnp
from jax import lax
from jax.experimental import pallas as pl
from jax.experimental.pallas import tpu as pltpu
from jax.experimental.pallas import tpu_sc as plsc

_EPS = 1e-08
_N_ROWS = 16384
_N_FEAT = 100
_N_CAT = 6
_BT = 2048
_NW = 32
_RPW = _N_ROWS // _NW
_NG = _RPW // 16

_LN2 = 0.6931471805599453


def _mm_body(w_ref, x_ref, b_ref, x1t_ref):
    x1t_ref[...] = lax.dot_general(w_ref[...], x_ref[...],
                                   (((1,), (1,)), ((), ()))) + b_ref[...]


def _log_poly(s):
    bits = lax.bitcast_convert_type(s, jnp.int32)
    e = ((bits >> 23) - 127).astype(jnp.float32)
    m = lax.bitcast_convert_type((bits & 0x7FFFFF) | 0x3F800000, jnp.float32)
    u = (m - 1.0) / (m + 1.0)
    u2 = u * u
    poly = 2.0 * u * (1.0 + u2 * (1.0 / 3.0 + u2 * (1.0 / 5.0
                      + u2 * (1.0 / 7.0 + u2 * (1.0 / 9.0)))))
    return poly + e * _LN2


def _sc_body(x1t_hbm, gt_hbm, elig_hbm, a_hbm, ent_hbm,
             x1_v, g_v, elig_v, a_v, ent_v):
    wid = lax.axis_index("s") * 2 + lax.axis_index("c")
    base = wid * _RPW
    pltpu.sync_copy(x1t_hbm.at[:, pl.ds(base, _RPW)], x1_v)
    pltpu.sync_copy(gt_hbm.at[:, pl.ds(base, _RPW)], g_v)

    lane = lax.iota(jnp.int32, 16)

    def group(i, ent_acc):
        off = i * 16
        l = [x1_v[k, pl.ds(off, 16)] for k in range(_N_CAT)]
        z = [l[k] + g_v[k, pl.ds(off, 16)] for k in range(_N_CAT)]

        m = l[0]
        for k in range(1, _N_CAT):
            m = jnp.maximum(m, l[k])
        e = [jnp.exp(l[k] - m) for k in range(_N_CAT)]
        s = e[0]
        for k in range(1, _N_CAT):
            s = s + e[k]
        r = m + _log_poly(s)
        logp = [l[k] - r for k in range(_N_CAT)]

        best = z[0]
        idx = jnp.zeros((16,), jnp.int32)
        for k in range(1, _N_CAT):
            take = z[k] > best
            best = jnp.maximum(best, z[k])
            idx = jnp.where(take, jnp.full((16,), k, jnp.int32), idx)

        rows = off + lane
        elig = jnp.zeros((16,), jnp.float32)
        ent = ent_acc
        inv_s = 1.0 / s
        for k in range(_N_CAT):
            a_k = jnp.where(idx == k, 1.0, 0.0)
            plsc.store_scatter(a_v, [rows, jnp.full((16,), k, jnp.int32)], a_k)
            elig = elig + a_k * logp[k]
            p_k = e[k] * inv_s
            ent = ent + (p_k + _EPS) * logp[k]
        elig_v[pl.ds(off, 16)] = elig
        return ent

    ent_acc = lax.fori_loop(0, _NG, group, jnp.zeros((16,), jnp.float32))
    ent_v[...] = -ent_acc

    pltpu.sync_copy(elig_v, elig_hbm.at[pl.ds(base, _RPW)])
    pltpu.sync_copy(a_v, a_hbm.at[pl.ds(base, _RPW), :])
    pltpu.sync_copy(ent_v, ent_hbm.at[pl.ds(wid * 16, 16)])


def _ent_body(p_ref, o_ref):
    o_ref[...] = jnp.sum(p_ref[...]).reshape(1, 1)


def kernel(x, W, b):
    with jax.ensure_compile_time_eval():
        g = jax.random.gumbel(jax.random.key(42), (_N_ROWS, _N_CAT),
                              jnp.float32)
        gt = g.T.copy()

    x1t = pl.pallas_call(
        _mm_body,
        grid=(_N_ROWS // _BT,),
        in_specs=[
            pl.BlockSpec((_N_CAT, _N_FEAT), lambda i: (0, 0)),
            pl.BlockSpec((_BT, _N_FEAT), lambda i: (i, 0)),
            pl.BlockSpec((_N_CAT, 1), lambda i: (0, 0)),
        ],
        out_specs=pl.BlockSpec((_N_CAT, _BT), lambda i: (0, i)),
        out_shape=jax.ShapeDtypeStruct((_N_CAT, _N_ROWS), jnp.float32),
    )(W, x, b[:, None])

    mesh = plsc.VectorSubcoreMesh(core_axis_name="c", subcore_axis_name="s")
    elig, a, ent_parts = pl.kernel(
        _sc_body,
        out_type=[
            jax.ShapeDtypeStruct((_N_ROWS,), jnp.float32),
            jax.ShapeDtypeStruct((_N_ROWS, _N_CAT), jnp.float32),
            jax.ShapeDtypeStruct((_NW * 16,), jnp.float32),
        ],
        mesh=mesh,
        compiler_params=pltpu.CompilerParams(needs_layout_passes=False),
        scratch_types=[
            pltpu.VMEM((_N_CAT, _RPW), jnp.float32),
            pltpu.VMEM((_N_CAT, _RPW), jnp.float32),
            pltpu.VMEM((_RPW,), jnp.float32),
            pltpu.VMEM((_RPW, _N_CAT), jnp.float32),
            pltpu.VMEM((16,), jnp.float32),
        ],
    )(x1t, gt)

    ent = pl.pallas_call(
        _ent_body,
        in_specs=[pl.BlockSpec((_NW * 16,), lambda: (0,))],
        out_specs=pl.BlockSpec((1, 1), lambda: (0, 0)),
        out_shape=jax.ShapeDtypeStruct((1, 1), jnp.float32),
    )(ent_parts)

    return elig, a, ent[0, 0]
